# Initial kernel scaffold; baseline (speedup 1.0000x reference)
import jax
import jax.numpy as jnp
from jax.experimental import pallas as pl


def kernel(protein_x, ligand_x, lig_W1, lig_b1, lig_W2, lig_b2, prot_W1, prot_b1, prot_W2, prot_b2, fc1_W, fc1_b, pkd_W, pkd_b, pki_W, pki_b, ba_W, ba_b, protein_edge_index, protein_batch, ligand_edge_index, ligand_batch):
    raise NotImplementedError("write your pallas kernel here")


# speedup vs baseline: 1.8715x; 1.8715x over previous
import functools

import jax
import jax.numpy as jnp
from jax.experimental import pallas as pl
from jax.experimental.pallas import tpu as pltpu

_G = 64
_H = 128
_EB = 1024
_RB = 512


def _round_up(x, m):
    return (x + m - 1) // m * m


def _deg_body(dst_ref, out_ref, *, nsteps):
    @pl.when(pl.program_id(0) == 0)
    def _():
        out_ref[...] = jnp.ones_like(out_ref)

    def body(i, carry):
        d = dst_ref[0, 0, i]
        out_ref[pl.ds(d, 1), :] = out_ref[pl.ds(d, 1), :] + 1.0
        return carry

    jax.lax.fori_loop(0, _EB, body, 0)


def _degree(dst_blocks, np_rows):
    nblk = dst_blocks.shape[0]
    return pl.pallas_call(
        functools.partial(_deg_body, nsteps=nblk),
        grid=(nblk,),
        in_specs=[
            pl.BlockSpec((1, 1, _EB), lambda i: (i, 0, 0),
                         memory_space=pltpu.SMEM),
        ],
        out_specs=pl.BlockSpec((np_rows, _H), lambda i: (0, 0)),
        out_shape=jax.ShapeDtypeStruct((np_rows, _H), jnp.float32),
    )(dst_blocks)


def _mm_body(x_ref, w_ref, dinv_ref, out_ref):
    h = jnp.dot(x_ref[...], w_ref[...], preferred_element_type=jnp.float32)
    out_ref[...] = h * dinv_ref[...]


def _matmul_scale(x, w, dinv):
    np_rows = x.shape[0]
    k = x.shape[1]
    grid = np_rows // _RB
    return pl.pallas_call(
        _mm_body,
        grid=(grid,),
        in_specs=[
            pl.BlockSpec((_RB, k), lambda i: (i, 0)),
            pl.BlockSpec((k, _H), lambda i: (0, 0)),
            pl.BlockSpec((_RB, _H), lambda i: (i, 0)),
        ],
        out_specs=pl.BlockSpec((_RB, _H), lambda i: (i, 0)),
        out_shape=jax.ShapeDtypeStruct((np_rows, _H), jnp.float32),
    )(x, w, dinv)


def _scatter_body(src_ref, dst_ref, h_ref, out_ref):
    @pl.when(pl.program_id(0) == 0)
    def _():
        out_ref[...] = h_ref[...]

    def body(i, carry):
        s = src_ref[0, 0, i]
        d = dst_ref[0, 0, i]
        out_ref[pl.ds(d, 1), :] = (
            out_ref[pl.ds(d, 1), :] + h_ref[pl.ds(s, 1), :])
        return carry

    jax.lax.fori_loop(0, _EB, body, 0, unroll=True)


def _edge_scatter(src_blocks, dst_blocks, h):
    np_rows = h.shape[0]
    nblk = src_blocks.shape[0]
    return pl.pallas_call(
        _scatter_body,
        grid=(nblk,),
        in_specs=[
            pl.BlockSpec((1, 1, _EB), lambda i: (i, 0, 0),
                         memory_space=pltpu.SMEM),
            pl.BlockSpec((1, 1, _EB), lambda i: (i, 0, 0),
                         memory_space=pltpu.SMEM),
            pl.BlockSpec((np_rows, _H), lambda i: (0, 0)),
        ],
        out_specs=pl.BlockSpec((np_rows, _H), lambda i: (0, 0)),
        out_shape=jax.ShapeDtypeStruct((np_rows, _H), jnp.float32),
    )(src_blocks, dst_blocks, h)


def _fin_body(x_ref, dinv_ref, b_ref, out_ref):
    out_ref[...] = jnp.maximum(x_ref[...] * dinv_ref[...] + b_ref[...], 0.0)


def _finalize(x, dinv, b):
    np_rows = x.shape[0]
    grid = np_rows // _RB
    return pl.pallas_call(
        _fin_body,
        grid=(grid,),
        in_specs=[
            pl.BlockSpec((_RB, _H), lambda i: (i, 0)),
            pl.BlockSpec((_RB, _H), lambda i: (i, 0)),
            pl.BlockSpec((1, _H), lambda i: (0, 0)),
        ],
        out_specs=pl.BlockSpec((_RB, _H), lambda i: (i, 0)),
        out_shape=jax.ShapeDtypeStruct((np_rows, _H), jnp.float32),
    )(x, dinv, b)


def _dinv_body(deg_ref, out_ref):
    d = deg_ref[...]
    out_ref[...] = jnp.where(d > 0, jax.lax.rsqrt(d), 0.0)


def _dinv(deg):
    np_rows = deg.shape[0]
    grid = np_rows // _RB
    return pl.pallas_call(
        _dinv_body,
        grid=(grid,),
        in_specs=[pl.BlockSpec((_RB, _H), lambda i: (i, 0))],
        out_specs=pl.BlockSpec((_RB, _H), lambda i: (i, 0)),
        out_shape=jax.ShapeDtypeStruct((np_rows, _H), jnp.float32),
    )(deg)


def _pool_body(batch_ref, x_ref, sum_ref, cnt_ref):
    @pl.when(pl.program_id(0) == 0)
    def _():
        sum_ref[...] = jnp.zeros_like(sum_ref)
        cnt_ref[...] = jnp.zeros_like(cnt_ref)

    b = batch_ref[0, 0, :]
    iota = jax.lax.broadcasted_iota(jnp.int32, (_G, _RB), 0)
    onehot = (iota == b[None, :]).astype(jnp.float32)
    sum_ref[...] += jnp.dot(onehot, x_ref[...],
                            preferred_element_type=jnp.float32)
    cnt_ref[...] += jnp.broadcast_to(
        jnp.sum(onehot, axis=1, keepdims=True), (_G, _H))


def _pool(batch_blocks, x):
    np_rows = x.shape[0]
    grid = np_rows // _RB
    return pl.pallas_call(
        _pool_body,
        grid=(grid,),
        in_specs=[
            pl.BlockSpec((1, 1, _RB), lambda i: (i, 0, 0)),
            pl.BlockSpec((_RB, _H), lambda i: (i, 0)),
        ],
        out_specs=[
            pl.BlockSpec((_G, _H), lambda i: (0, 0)),
            pl.BlockSpec((_G, _H), lambda i: (0, 0)),
        ],
        out_shape=[
            jax.ShapeDtypeStruct((_G, _H), jnp.float32),
            jax.ShapeDtypeStruct((_G, _H), jnp.float32),
        ],
    )(batch_blocks, x)


def _head_body(ps_ref, pc_ref, ls_ref, lc_ref, w1p_ref, w1l_ref, b1_ref,
               wkd_ref, bkd_ref, wki_ref, bki_ref, wba_ref, bba_ref,
               pkd_ref, pki_ref, ba_ref):
    pe = ps_ref[...] / jnp.maximum(pc_ref[...], 1.0)
    le = ls_ref[...] / jnp.maximum(lc_ref[...], 1.0)
    c = (jnp.dot(pe, w1p_ref[...], preferred_element_type=jnp.float32)
         + jnp.dot(le, w1l_ref[...], preferred_element_type=jnp.float32)
         + b1_ref[...])
    c = jnp.maximum(c, 0.0)
    pkd_ref[...] = jnp.dot(c, wkd_ref[...],
                           preferred_element_type=jnp.float32) + bkd_ref[...]
    pki_ref[...] = jnp.dot(c, wki_ref[...],
                           preferred_element_type=jnp.float32) + bki_ref[...]
    ba_ref[...] = jnp.dot(c, wba_ref[...],
                          preferred_element_type=jnp.float32) + bba_ref[...]


def _heads(ps, pc, ls, lc, w1p, w1l, b1, wkd, bkd, wki, bki, wba, bba):
    full = lambda s: pl.BlockSpec(s, lambda: tuple(0 for _ in s))
    return pl.pallas_call(
        _head_body,
        in_specs=[
            full((_G, _H)), full((_G, _H)), full((_G, _H)), full((_G, _H)),
            full((_H, _H)), full((_H, _H)), full((1, _H)),
            full((_H, 1)), full((1, 1)),
            full((_H, 1)), full((1, 1)),
            full((_H, 1)), full((1, 1)),
        ],
        out_specs=[full((_G, 1)), full((_G, 1)), full((_G, 1))],
        out_shape=[
            jax.ShapeDtypeStruct((_G, 1), jnp.float32),
            jax.ShapeDtypeStruct((_G, 1), jnp.float32),
            jax.ShapeDtypeStruct((_G, 1), jnp.float32),
        ],
    )(ps, pc, ls, lc, w1p, w1l, b1, wkd, bkd, wki, bki, wba, bba)


def _prep_edges(edge_index, n):
    e = edge_index.shape[1]
    e_pad = _round_up(e, _EB)
    src = jnp.full((e_pad,), 0, jnp.int32).at[:e].set(edge_index[0])
    dst = jnp.full((e_pad,), n, jnp.int32).at[:e].set(edge_index[1])
    nblk = e_pad // _EB
    return (src.reshape(nblk, 1, _EB), dst.reshape(nblk, 1, _EB))


def _gcn_stack(x, edge_index, batch, w1, b1, w2, b2):
    n, f = x.shape
    np_rows = _round_up(n + 1, _RB)
    xp = jnp.zeros((np_rows, f), jnp.float32).at[:n].set(x)
    src_b, dst_b = _prep_edges(edge_index, n)

    deg = _degree(dst_b, np_rows)
    dinv = _dinv(deg)

    h1 = _matmul_scale(xp, w1, dinv)
    s1 = _edge_scatter(src_b, dst_b, h1)
    y1 = _finalize(s1, dinv, b1.reshape(1, _H))

    h2 = _matmul_scale(y1, w2, dinv)
    s2 = _edge_scatter(src_b, dst_b, h2)
    y2 = _finalize(s2, dinv, b2.reshape(1, _H))

    bpad = jnp.full((np_rows,), _G, jnp.int32).at[:n].set(batch)
    bblocks = bpad.reshape(np_rows // _RB, 1, _RB)
    return _pool(bblocks, y2)


def kernel(protein_x, ligand_x, lig_W1, lig_b1, lig_W2, lig_b2, prot_W1,
           prot_b1, prot_W2, prot_b2, fc1_W, fc1_b, pkd_W, pkd_b, pki_W,
           pki_b, ba_W, ba_b, protein_edge_index, protein_batch,
           ligand_edge_index, ligand_batch):
    ls, lc = _gcn_stack(ligand_x, ligand_edge_index, ligand_batch,
                        lig_W1, lig_b1, lig_W2, lig_b2)
    ps, pc = _gcn_stack(protein_x, protein_edge_index, protein_batch,
                        prot_W1, prot_b1, prot_W2, prot_b2)
    pkd, pki, ba = _heads(
        ps, pc, ls, lc,
        fc1_W[:_H], fc1_W[_H:], fc1_b.reshape(1, _H),
        pkd_W, pkd_b.reshape(1, 1),
        pki_W, pki_b.reshape(1, 1),
        ba_W, ba_b.reshape(1, 1),
    )
    return (pkd, pki, ba)


# speedup vs baseline: 2.4036x; 1.2843x over previous
import functools

import jax
import jax.numpy as jnp
from jax.experimental import pallas as pl
from jax.experimental.pallas import tpu as pltpu

_G = 64
_H = 128
_EB = 1024
_RB = 512


def _round_up(x, m):
    return (x + m - 1) // m * m


def _deg_body(dst_ref, out_ref, *, nsteps):
    @pl.when(pl.program_id(0) == 0)
    def _():
        out_ref[...] = jnp.ones_like(out_ref)

    def body(i, carry):
        d = dst_ref[0, 0, i]
        out_ref[pl.ds(d, 1), :] = out_ref[pl.ds(d, 1), :] + 1.0
        return carry

    jax.lax.fori_loop(0, _EB, body, 0, unroll=8)


def _degree(dst_blocks, np_rows):
    nblk = dst_blocks.shape[0]
    return pl.pallas_call(
        functools.partial(_deg_body, nsteps=nblk),
        grid=(nblk,),
        in_specs=[
            pl.BlockSpec((1, 1, _EB), lambda i: (i, 0, 0),
                         memory_space=pltpu.SMEM),
        ],
        out_specs=pl.BlockSpec((np_rows, _H), lambda i: (0, 0)),
        out_shape=jax.ShapeDtypeStruct((np_rows, _H), jnp.float32),
    )(dst_blocks)


def _mm_body(x_ref, w_ref, dinv_ref, out_ref):
    h = jnp.dot(x_ref[...], w_ref[...], preferred_element_type=jnp.float32)
    out_ref[...] = h * dinv_ref[...]


def _matmul_scale(x, w, dinv):
    np_rows = x.shape[0]
    k = x.shape[1]
    grid = np_rows // _RB
    return pl.pallas_call(
        _mm_body,
        grid=(grid,),
        in_specs=[
            pl.BlockSpec((_RB, k), lambda i: (i, 0)),
            pl.BlockSpec((k, _H), lambda i: (0, 0)),
            pl.BlockSpec((_RB, _H), lambda i: (i, 0)),
        ],
        out_specs=pl.BlockSpec((_RB, _H), lambda i: (i, 0)),
        out_shape=jax.ShapeDtypeStruct((np_rows, _H), jnp.float32),
    )(x, w, dinv)


def _scatter_body(src_ref, dst_ref, h_ref, out_ref):
    @pl.when(pl.program_id(0) == 0)
    def _():
        out_ref[...] = h_ref[...]

    def body(i, carry):
        s = src_ref[0, 0, i]
        d = dst_ref[0, 0, i]
        out_ref[pl.ds(d, 1), :] = (
            out_ref[pl.ds(d, 1), :] + h_ref[pl.ds(s, 1), :])
        return carry

    jax.lax.fori_loop(0, _EB, body, 0, unroll=True)


def _edge_scatter(src_blocks, dst_blocks, h):
    np_rows = h.shape[0]
    nblk = src_blocks.shape[0]
    return pl.pallas_call(
        _scatter_body,
        grid=(nblk,),
        in_specs=[
            pl.BlockSpec((1, 1, _EB), lambda i: (i, 0, 0),
                         memory_space=pltpu.SMEM),
            pl.BlockSpec((1, 1, _EB), lambda i: (i, 0, 0),
                         memory_space=pltpu.SMEM),
            pl.BlockSpec((np_rows, _H), lambda i: (0, 0)),
        ],
        out_specs=pl.BlockSpec((np_rows, _H), lambda i: (0, 0)),
        out_shape=jax.ShapeDtypeStruct((np_rows, _H), jnp.float32),
    )(src_blocks, dst_blocks, h)


def _fin_body(x_ref, dinv_ref, b_ref, out_ref):
    out_ref[...] = jnp.maximum(x_ref[...] * dinv_ref[...] + b_ref[...], 0.0)


def _finalize(x, dinv, b):
    np_rows = x.shape[0]
    grid = np_rows // _RB
    return pl.pallas_call(
        _fin_body,
        grid=(grid,),
        in_specs=[
            pl.BlockSpec((_RB, _H), lambda i: (i, 0)),
            pl.BlockSpec((_RB, _H), lambda i: (i, 0)),
            pl.BlockSpec((1, _H), lambda i: (0, 0)),
        ],
        out_specs=pl.BlockSpec((_RB, _H), lambda i: (i, 0)),
        out_shape=jax.ShapeDtypeStruct((np_rows, _H), jnp.float32),
    )(x, dinv, b)


def _dinv_body(deg_ref, out_ref):
    d = deg_ref[...]
    out_ref[...] = jnp.where(d > 0, jax.lax.rsqrt(d), 0.0)


def _dinv(deg):
    np_rows = deg.shape[0]
    grid = np_rows // _RB
    return pl.pallas_call(
        _dinv_body,
        grid=(grid,),
        in_specs=[pl.BlockSpec((_RB, _H), lambda i: (i, 0))],
        out_specs=pl.BlockSpec((_RB, _H), lambda i: (i, 0)),
        out_shape=jax.ShapeDtypeStruct((np_rows, _H), jnp.float32),
    )(deg)


def _pool_body(batch_ref, x_ref, sum_ref, cnt_ref):
    @pl.when(pl.program_id(0) == 0)
    def _():
        sum_ref[...] = jnp.zeros_like(sum_ref)
        cnt_ref[...] = jnp.zeros_like(cnt_ref)

    b = batch_ref[0, 0, :]
    iota = jax.lax.broadcasted_iota(jnp.int32, (_G, _RB), 0)
    onehot = (iota == b[None, :]).astype(jnp.float32)
    sum_ref[...] += jnp.dot(onehot, x_ref[...],
                            preferred_element_type=jnp.float32)
    cnt_ref[...] += jnp.broadcast_to(
        jnp.sum(onehot, axis=1, keepdims=True), (_G, _H))


def _pool(batch_blocks, x):
    np_rows = x.shape[0]
    grid = np_rows // _RB
    return pl.pallas_call(
        _pool_body,
        grid=(grid,),
        in_specs=[
            pl.BlockSpec((1, 1, _RB), lambda i: (i, 0, 0)),
            pl.BlockSpec((_RB, _H), lambda i: (i, 0)),
        ],
        out_specs=[
            pl.BlockSpec((_G, _H), lambda i: (0, 0)),
            pl.BlockSpec((_G, _H), lambda i: (0, 0)),
        ],
        out_shape=[
            jax.ShapeDtypeStruct((_G, _H), jnp.float32),
            jax.ShapeDtypeStruct((_G, _H), jnp.float32),
        ],
    )(batch_blocks, x)


def _head_body(ps_ref, pc_ref, ls_ref, lc_ref, w1p_ref, w1l_ref, b1_ref,
               wkd_ref, bkd_ref, wki_ref, bki_ref, wba_ref, bba_ref,
               pkd_ref, pki_ref, ba_ref):
    pe = ps_ref[...] / jnp.maximum(pc_ref[...], 1.0)
    le = ls_ref[...] / jnp.maximum(lc_ref[...], 1.0)
    c = (jnp.dot(pe, w1p_ref[...], preferred_element_type=jnp.float32)
         + jnp.dot(le, w1l_ref[...], preferred_element_type=jnp.float32)
         + b1_ref[...])
    c = jnp.maximum(c, 0.0)
    pkd_ref[...] = jnp.dot(c, wkd_ref[...],
                           preferred_element_type=jnp.float32) + bkd_ref[...]
    pki_ref[...] = jnp.dot(c, wki_ref[...],
                           preferred_element_type=jnp.float32) + bki_ref[...]
    ba_ref[...] = jnp.dot(c, wba_ref[...],
                          preferred_element_type=jnp.float32) + bba_ref[...]


def _heads(ps, pc, ls, lc, w1p, w1l, b1, wkd, bkd, wki, bki, wba, bba):
    full = lambda s: pl.BlockSpec(s, lambda: tuple(0 for _ in s))
    return pl.pallas_call(
        _head_body,
        in_specs=[
            full((_G, _H)), full((_G, _H)), full((_G, _H)), full((_G, _H)),
            full((_H, _H)), full((_H, _H)), full((1, _H)),
            full((_H, 1)), full((1, 1)),
            full((_H, 1)), full((1, 1)),
            full((_H, 1)), full((1, 1)),
        ],
        out_specs=[full((_G, 1)), full((_G, 1)), full((_G, 1))],
        out_shape=[
            jax.ShapeDtypeStruct((_G, 1), jnp.float32),
            jax.ShapeDtypeStruct((_G, 1), jnp.float32),
            jax.ShapeDtypeStruct((_G, 1), jnp.float32),
        ],
    )(ps, pc, ls, lc, w1p, w1l, b1, wkd, bkd, wki, bki, wba, bba)


def _prep_edges(edge_index, n):
    e = edge_index.shape[1]
    e_pad = _round_up(e, _EB)
    src = jnp.full((e_pad,), 0, jnp.int32).at[:e].set(edge_index[0])
    dst = jnp.full((e_pad,), n, jnp.int32).at[:e].set(edge_index[1])
    nblk = e_pad // _EB
    return (src.reshape(nblk, 1, _EB), dst.reshape(nblk, 1, _EB))


def _gcn_stack(x, edge_index, batch, w1, b1, w2, b2):
    n, f = x.shape
    np_rows = _round_up(n + 1, _RB)
    xp = jnp.zeros((np_rows, f), jnp.float32).at[:n].set(x)
    src_b, dst_b = _prep_edges(edge_index, n)

    deg = _degree(dst_b, np_rows)
    dinv = _dinv(deg)

    h1 = _matmul_scale(xp, w1, dinv)
    s1 = _edge_scatter(src_b, dst_b, h1)
    y1 = _finalize(s1, dinv, b1.reshape(1, _H))

    h2 = _matmul_scale(y1, w2, dinv)
    s2 = _edge_scatter(src_b, dst_b, h2)
    y2 = _finalize(s2, dinv, b2.reshape(1, _H))

    bpad = jnp.full((np_rows,), _G, jnp.int32).at[:n].set(batch)
    bblocks = bpad.reshape(np_rows // _RB, 1, _RB)
    return _pool(bblocks, y2)


def kernel(protein_x, ligand_x, lig_W1, lig_b1, lig_W2, lig_b2, prot_W1,
           prot_b1, prot_W2, prot_b2, fc1_W, fc1_b, pkd_W, pkd_b, pki_W,
           pki_b, ba_W, ba_b, protein_edge_index, protein_batch,
           ligand_edge_index, ligand_batch):
    ls, lc = _gcn_stack(ligand_x, ligand_edge_index, ligand_batch,
                        lig_W1, lig_b1, lig_W2, lig_b2)
    ps, pc = _gcn_stack(protein_x, protein_edge_index, protein_batch,
                        prot_W1, prot_b1, prot_W2, prot_b2)
    pkd, pki, ba = _heads(
        ps, pc, ls, lc,
        fc1_W[:_H], fc1_W[_H:], fc1_b.reshape(1, _H),
        pkd_W, pkd_b.reshape(1, 1),
        pki_W, pki_b.reshape(1, 1),
        ba_W, ba_b.reshape(1, 1),
    )
    return (pkd, pki, ba)


# speedup vs baseline: 2.4683x; 1.0269x over previous
import functools

import jax
import jax.numpy as jnp
from jax.experimental import pallas as pl
from jax.experimental.pallas import tpu as pltpu

_G = 64
_H = 128
_EB = 1024
_RB = 512


def _round_up(x, m):
    return (x + m - 1) // m * m


def _deg_body(dst_ref, out_ref, *, nsteps):
    @pl.when(pl.program_id(0) == 0)
    def _():
        out_ref[...] = jnp.ones_like(out_ref)

    def body(i, carry):
        d = dst_ref[0, 0, i]
        out_ref[pl.ds(d, 1), :] = out_ref[pl.ds(d, 1), :] + 1.0
        return carry

    jax.lax.fori_loop(0, _EB, body, 0, unroll=True)


def _degree(dst_blocks, np_rows):
    nblk = dst_blocks.shape[0]
    return pl.pallas_call(
        functools.partial(_deg_body, nsteps=nblk),
        grid=(nblk,),
        in_specs=[
            pl.BlockSpec((1, 1, _EB), lambda i: (i, 0, 0),
                         memory_space=pltpu.SMEM),
        ],
        out_specs=pl.BlockSpec((np_rows, _H), lambda i: (0, 0)),
        out_shape=jax.ShapeDtypeStruct((np_rows, _H), jnp.float32),
    )(dst_blocks)


def _mm_body(x_ref, w_ref, dinv_ref, out_ref):
    h = jnp.dot(x_ref[...], w_ref[...], preferred_element_type=jnp.float32)
    out_ref[...] = h * dinv_ref[...]


def _matmul_scale(x, w, dinv):
    np_rows = x.shape[0]
    k = x.shape[1]
    grid = np_rows // _RB
    return pl.pallas_call(
        _mm_body,
        grid=(grid,),
        in_specs=[
            pl.BlockSpec((_RB, k), lambda i: (i, 0)),
            pl.BlockSpec((k, _H), lambda i: (0, 0)),
            pl.BlockSpec((_RB, _H), lambda i: (i, 0)),
        ],
        out_specs=pl.BlockSpec((_RB, _H), lambda i: (i, 0)),
        out_shape=jax.ShapeDtypeStruct((np_rows, _H), jnp.float32),
    )(x, w, dinv)


def _scatter_body(src_ref, dst_ref, h_ref, out_ref):
    @pl.when(pl.program_id(0) == 0)
    def _():
        out_ref[...] = h_ref[...]

    def body(i, carry):
        s = src_ref[0, 0, i]
        d = dst_ref[0, 0, i]
        out_ref[pl.ds(d, 1), :] = (
            out_ref[pl.ds(d, 1), :] + h_ref[pl.ds(s, 1), :])
        return carry

    jax.lax.fori_loop(0, _EB, body, 0, unroll=True)


def _edge_scatter(src_blocks, dst_blocks, h):
    np_rows = h.shape[0]
    nblk = src_blocks.shape[0]
    return pl.pallas_call(
        _scatter_body,
        grid=(nblk,),
        in_specs=[
            pl.BlockSpec((1, 1, _EB), lambda i: (i, 0, 0),
                         memory_space=pltpu.SMEM),
            pl.BlockSpec((1, 1, _EB), lambda i: (i, 0, 0),
                         memory_space=pltpu.SMEM),
            pl.BlockSpec((np_rows, _H), lambda i: (0, 0)),
        ],
        out_specs=pl.BlockSpec((np_rows, _H), lambda i: (0, 0)),
        out_shape=jax.ShapeDtypeStruct((np_rows, _H), jnp.float32),
    )(src_blocks, dst_blocks, h)


def _fin_body(x_ref, dinv_ref, b_ref, out_ref):
    out_ref[...] = jnp.maximum(x_ref[...] * dinv_ref[...] + b_ref[...], 0.0)


def _finalize(x, dinv, b):
    np_rows = x.shape[0]
    grid = np_rows // _RB
    return pl.pallas_call(
        _fin_body,
        grid=(grid,),
        in_specs=[
            pl.BlockSpec((_RB, _H), lambda i: (i, 0)),
            pl.BlockSpec((_RB, _H), lambda i: (i, 0)),
            pl.BlockSpec((1, _H), lambda i: (0, 0)),
        ],
        out_specs=pl.BlockSpec((_RB, _H), lambda i: (i, 0)),
        out_shape=jax.ShapeDtypeStruct((np_rows, _H), jnp.float32),
    )(x, dinv, b)


def _dinv_body(deg_ref, out_ref):
    d = deg_ref[...]
    out_ref[...] = jnp.where(d > 0, jax.lax.rsqrt(d), 0.0)


def _dinv(deg):
    np_rows = deg.shape[0]
    grid = np_rows // _RB
    return pl.pallas_call(
        _dinv_body,
        grid=(grid,),
        in_specs=[pl.BlockSpec((_RB, _H), lambda i: (i, 0))],
        out_specs=pl.BlockSpec((_RB, _H), lambda i: (i, 0)),
        out_shape=jax.ShapeDtypeStruct((np_rows, _H), jnp.float32),
    )(deg)


def _pool_body(batch_ref, x_ref, sum_ref, cnt_ref):
    @pl.when(pl.program_id(0) == 0)
    def _():
        sum_ref[...] = jnp.zeros_like(sum_ref)
        cnt_ref[...] = jnp.zeros_like(cnt_ref)

    b = batch_ref[0, 0, :]
    iota = jax.lax.broadcasted_iota(jnp.int32, (_G, _RB), 0)
    onehot = (iota == b[None, :]).astype(jnp.float32)
    sum_ref[...] += jnp.dot(onehot, x_ref[...],
                            preferred_element_type=jnp.float32)
    cnt_ref[...] += jnp.broadcast_to(
        jnp.sum(onehot, axis=1, keepdims=True), (_G, _H))


def _pool(batch_blocks, x):
    np_rows = x.shape[0]
    grid = np_rows // _RB
    return pl.pallas_call(
        _pool_body,
        grid=(grid,),
        in_specs=[
            pl.BlockSpec((1, 1, _RB), lambda i: (i, 0, 0)),
            pl.BlockSpec((_RB, _H), lambda i: (i, 0)),
        ],
        out_specs=[
            pl.BlockSpec((_G, _H), lambda i: (0, 0)),
            pl.BlockSpec((_G, _H), lambda i: (0, 0)),
        ],
        out_shape=[
            jax.ShapeDtypeStruct((_G, _H), jnp.float32),
            jax.ShapeDtypeStruct((_G, _H), jnp.float32),
        ],
    )(batch_blocks, x)


def _head_body(ps_ref, pc_ref, ls_ref, lc_ref, w1p_ref, w1l_ref, b1_ref,
               wkd_ref, bkd_ref, wki_ref, bki_ref, wba_ref, bba_ref,
               pkd_ref, pki_ref, ba_ref):
    pe = ps_ref[...] / jnp.maximum(pc_ref[...], 1.0)
    le = ls_ref[...] / jnp.maximum(lc_ref[...], 1.0)
    c = (jnp.dot(pe, w1p_ref[...], preferred_element_type=jnp.float32)
         + jnp.dot(le, w1l_ref[...], preferred_element_type=jnp.float32)
         + b1_ref[...])
    c = jnp.maximum(c, 0.0)
    pkd_ref[...] = jnp.dot(c, wkd_ref[...],
                           preferred_element_type=jnp.float32) + bkd_ref[...]
    pki_ref[...] = jnp.dot(c, wki_ref[...],
                           preferred_element_type=jnp.float32) + bki_ref[...]
    ba_ref[...] = jnp.dot(c, wba_ref[...],
                          preferred_element_type=jnp.float32) + bba_ref[...]


def _heads(ps, pc, ls, lc, w1p, w1l, b1, wkd, bkd, wki, bki, wba, bba):
    full = lambda s: pl.BlockSpec(s, lambda: tuple(0 for _ in s))
    return pl.pallas_call(
        _head_body,
        in_specs=[
            full((_G, _H)), full((_G, _H)), full((_G, _H)), full((_G, _H)),
            full((_H, _H)), full((_H, _H)), full((1, _H)),
            full((_H, 1)), full((1, 1)),
            full((_H, 1)), full((1, 1)),
            full((_H, 1)), full((1, 1)),
        ],
        out_specs=[full((_G, 1)), full((_G, 1)), full((_G, 1))],
        out_shape=[
            jax.ShapeDtypeStruct((_G, 1), jnp.float32),
            jax.ShapeDtypeStruct((_G, 1), jnp.float32),
            jax.ShapeDtypeStruct((_G, 1), jnp.float32),
        ],
    )(ps, pc, ls, lc, w1p, w1l, b1, wkd, bkd, wki, bki, wba, bba)


def _prep_edges(edge_index, n):
    e = edge_index.shape[1]
    e_pad = _round_up(e, _EB)
    src = jnp.full((e_pad,), 0, jnp.int32).at[:e].set(edge_index[0])
    dst = jnp.full((e_pad,), n, jnp.int32).at[:e].set(edge_index[1])
    nblk = e_pad // _EB
    return (src.reshape(nblk, 1, _EB), dst.reshape(nblk, 1, _EB))


def _gcn_stack(x, edge_index, batch, w1, b1, w2, b2):
    n, f = x.shape
    np_rows = _round_up(n + 1, _RB)
    xp = jnp.zeros((np_rows, f), jnp.float32).at[:n].set(x)
    src_b, dst_b = _prep_edges(edge_index, n)

    deg = _degree(dst_b, np_rows)
    dinv = _dinv(deg)

    h1 = _matmul_scale(xp, w1, dinv)
    s1 = _edge_scatter(src_b, dst_b, h1)
    y1 = _finalize(s1, dinv, b1.reshape(1, _H))

    h2 = _matmul_scale(y1, w2, dinv)
    s2 = _edge_scatter(src_b, dst_b, h2)
    y2 = _finalize(s2, dinv, b2.reshape(1, _H))

    bpad = jnp.full((np_rows,), _G, jnp.int32).at[:n].set(batch)
    bblocks = bpad.reshape(np_rows // _RB, 1, _RB)
    return _pool(bblocks, y2)


def kernel(protein_x, ligand_x, lig_W1, lig_b1, lig_W2, lig_b2, prot_W1,
           prot_b1, prot_W2, prot_b2, fc1_W, fc1_b, pkd_W, pkd_b, pki_W,
           pki_b, ba_W, ba_b, protein_edge_index, protein_batch,
           ligand_edge_index, ligand_batch):
    ls, lc = _gcn_stack(ligand_x, ligand_edge_index, ligand_batch,
                        lig_W1, lig_b1, lig_W2, lig_b2)
    ps, pc = _gcn_stack(protein_x, protein_edge_index, protein_batch,
                        prot_W1, prot_b1, prot_W2, prot_b2)
    pkd, pki, ba = _heads(
        ps, pc, ls, lc,
        fc1_W[:_H], fc1_W[_H:], fc1_b.reshape(1, _H),
        pkd_W, pkd_b.reshape(1, 1),
        pki_W, pki_b.reshape(1, 1),
        ba_W, ba_b.reshape(1, 1),
    )
    return (pkd, pki, ba)
